# Initial kernel scaffold; baseline (speedup 1.0000x reference)
import jax
import jax.numpy as jnp
from jax.experimental import pallas as pl


def kernel(input_imgs, token_type_ids, word_emb, pos_emb, type_emb, ln_gamma, ln_beta):
    raise NotImplementedError("write your pallas kernel here")


# speedup vs baseline: 4.3301x; 4.3301x over previous
import jax
import jax.numpy as jnp
from jax.experimental import pallas as pl

_LN_EPS = 1e-12
_CLS_ID = 101
_SEP_ID = 102


def _fused_body(imgs_ref, tt_ref, pos_ref, type_ref, cls_ref, sep_ref,
                gamma_ref, beta_ref, out_ref):
    t0 = type_ref[0:1, :]
    td = type_ref[1:2, :] - t0
    gamma = gamma_ref[0:1, :]
    beta = beta_ref[0:1, :]
    tt = tt_ref[...]

    def ln_store(x, s_lo, s_hi):
        mean = jnp.mean(x, axis=-1, keepdims=True)
        xc = x - mean
        var = jnp.mean(xc * xc, axis=-1, keepdims=True)
        y = xc * jax.lax.rsqrt(var + _LN_EPS)
        out_ref[:, s_lo:s_hi, :] = y * gamma[None] + beta[None]

    x_cls = (cls_ref[0:1, :] + pos_ref[0:1, :] + t0)[None] \
        + tt[:, 0:1, None] * td[None]
    ln_store(x_cls, 0, 1)

    x_mid = imgs_ref[...] + (pos_ref[1:51, :] + t0)[None] \
        + tt[:, 1:51, None] * td[None]
    ln_store(x_mid, 1, 51)

    x_sep = (sep_ref[0:1, :] + pos_ref[51:52, :] + t0)[None] \
        + tt[:, 51:52, None] * td[None]
    ln_store(x_sep, 51, 52)


def kernel(input_imgs, token_type_ids, word_emb, pos_emb, type_emb, ln_gamma, ln_beta):
    bsz, num_img, hidden = input_imgs.shape
    seq = num_img + 2
    tb = 64
    grid = (bsz // tb,)

    tt_f = token_type_ids.astype(jnp.float32)
    cls_row = jax.lax.slice(word_emb, (_CLS_ID, 0), (_CLS_ID + 1, hidden))
    sep_row = jax.lax.slice(word_emb, (_SEP_ID, 0), (_SEP_ID + 1, hidden))
    pos_slice = pos_emb[:seq]
    gamma2 = ln_gamma.reshape(1, hidden)
    beta2 = ln_beta.reshape(1, hidden)

    return pl.pallas_call(
        _fused_body,
        grid=grid,
        in_specs=[
            pl.BlockSpec((tb, num_img, hidden), lambda i: (i, 0, 0)),
            pl.BlockSpec((tb, seq), lambda i: (i, 0)),
            pl.BlockSpec((seq, hidden), lambda i: (0, 0)),
            pl.BlockSpec((2, hidden), lambda i: (0, 0)),
            pl.BlockSpec((1, hidden), lambda i: (0, 0)),
            pl.BlockSpec((1, hidden), lambda i: (0, 0)),
            pl.BlockSpec((1, hidden), lambda i: (0, 0)),
            pl.BlockSpec((1, hidden), lambda i: (0, 0)),
        ],
        out_specs=pl.BlockSpec((tb, seq, hidden), lambda i: (i, 0, 0)),
        out_shape=jax.ShapeDtypeStruct((bsz, seq, hidden), jnp.float32),
    )(input_imgs, tt_f, pos_slice, type_emb, cls_row, sep_row, gamma2, beta2)


# speedup vs baseline: 4.3478x; 1.0041x over previous
import jax
import jax.numpy as jnp
from jax.experimental import pallas as pl
from jax.experimental.pallas import tpu as pltpu

_LN_EPS = 1e-12
_CLS_ID = 101
_SEP_ID = 102


def _fused_body(imgs_ref, tt_ref, pos_ref, type_ref, cls_ref, sep_ref,
                gamma_ref, beta_ref, out_ref):
    t0 = type_ref[0:1, :]
    td = type_ref[1:2, :] - t0
    gamma = gamma_ref[0:1, :]
    beta = beta_ref[0:1, :]
    tt = tt_ref[...]

    def ln_store(x, s_lo, s_hi):
        mean = jnp.mean(x, axis=-1, keepdims=True)
        xc = x - mean
        var = jnp.mean(xc * xc, axis=-1, keepdims=True)
        y = xc * jax.lax.rsqrt(var + _LN_EPS)
        out_ref[:, s_lo:s_hi, :] = y * gamma[None] + beta[None]

    x_cls = (cls_ref[0:1, :] + pos_ref[0:1, :] + t0)[None] \
        + tt[:, 0:1, None] * td[None]
    ln_store(x_cls, 0, 1)

    x_mid = imgs_ref[...] + (pos_ref[1:51, :] + t0)[None] \
        + tt[:, 1:51, None] * td[None]
    ln_store(x_mid, 1, 51)

    x_sep = (sep_ref[0:1, :] + pos_ref[51:52, :] + t0)[None] \
        + tt[:, 51:52, None] * td[None]
    ln_store(x_sep, 51, 52)


def kernel(input_imgs, token_type_ids, word_emb, pos_emb, type_emb, ln_gamma, ln_beta):
    bsz, num_img, hidden = input_imgs.shape
    seq = num_img + 2
    tb = 64
    grid = (bsz // tb,)

    tt_f = token_type_ids.astype(jnp.float32)
    cls_row = jax.lax.slice(word_emb, (_CLS_ID, 0), (_CLS_ID + 1, hidden))
    sep_row = jax.lax.slice(word_emb, (_SEP_ID, 0), (_SEP_ID + 1, hidden))
    pos_slice = pos_emb[:seq]
    gamma2 = ln_gamma.reshape(1, hidden)
    beta2 = ln_beta.reshape(1, hidden)

    return pl.pallas_call(
        _fused_body,
        grid=grid,
        in_specs=[
            pl.BlockSpec((tb, num_img, hidden), lambda i: (i, 0, 0)),
            pl.BlockSpec((tb, seq), lambda i: (i, 0)),
            pl.BlockSpec((seq, hidden), lambda i: (0, 0)),
            pl.BlockSpec((2, hidden), lambda i: (0, 0)),
            pl.BlockSpec((1, hidden), lambda i: (0, 0)),
            pl.BlockSpec((1, hidden), lambda i: (0, 0)),
            pl.BlockSpec((1, hidden), lambda i: (0, 0)),
            pl.BlockSpec((1, hidden), lambda i: (0, 0)),
        ],
        out_specs=pl.BlockSpec((tb, seq, hidden), lambda i: (i, 0, 0)),
        out_shape=jax.ShapeDtypeStruct((bsz, seq, hidden), jnp.float32),
        compiler_params=pltpu.CompilerParams(
            dimension_semantics=("parallel",),
        ),
    )(input_imgs, tt_f, pos_slice, type_emb, cls_row, sep_row, gamma2, beta2)
